# baseline (speedup 1.0000x reference)
import functools
import math

import jax
import jax.numpy as jnp
from jax import lax
from jax.experimental import pallas as pl
from jax.experimental.pallas import tpu as pltpu
from jax.experimental.pallas import tpu_sc as plsc

_NUM_CORES = 2
_NUM_SUBCORES = 16
_NW = _NUM_CORES * _NUM_SUBCORES
_LANES = 16

_SUB_F = 640
_ROWS_PER_SUB = 3 * _SUB_F
_IDX_BLK = 128
_NBLK = _ROWS_PER_SUB // _IDX_BLK
_GROUPS = _SUB_F // _LANES


def _rsqrt(x, iters):
    i = plsc.bitcast(x, jnp.int32)
    r = plsc.bitcast(jnp.int32(0x5F3759DF) - (i >> 1), jnp.float32)
    for _ in range(iters):
        r = r * (1.5 - 0.5 * x * r * r)
    return r


def _acos(x):
    ax = jnp.minimum(jnp.abs(x), 1.0)
    t = 1.0 - ax
    s = t * _rsqrt(t, 2)
    p = jnp.float32(-0.0187293)
    p = p * ax + 0.0742610
    p = p * ax - 0.2121144
    p = p * ax + 1.5707288
    a = s * p
    return jnp.where(x >= 0.0, a, jnp.float32(math.pi) - a)


def kernel(fs, faces):
    batch, num_v, _ = fs.shape
    num_f = faces.shape[0]
    cols = batch * 3

    per_round_f = _SUB_F * _NW
    nsub = -(-num_f // per_round_f)
    f_pad = nsub * per_round_f
    h_pad = 3 * f_pad
    tile_h = nsub * _ROWS_PER_SUB

    packed = jnp.concatenate(
        [fs[b] for b in range(batch)]
        + [jnp.zeros((num_v, _LANES - cols), jnp.float32)],
        axis=1,
    )
    tails = jnp.pad(faces.astype(jnp.int32).reshape(-1), (0, h_pad - 3 * num_f))

    @functools.partial(
        pl.kernel,
        out_type=jax.ShapeDtypeStruct((batch * h_pad,), jnp.float32),
        mesh=plsc.VectorSubcoreMesh(core_axis_name="c", subcore_axis_name="s"),
        scratch_types=[
            pltpu.VMEM((_ROWS_PER_SUB,), jnp.int32),
            pltpu.VMEM((_ROWS_PER_SUB,), jnp.int32),
            pltpu.VMEM((_ROWS_PER_SUB, _LANES), jnp.float32),
            pltpu.VMEM((_ROWS_PER_SUB, _LANES), jnp.float32),
            pltpu.VMEM((batch, tile_h), jnp.float32),
            pltpu.SemaphoreType.DMA,
            pltpu.SemaphoreType.DMA,
        ],
        compiler_params=pltpu.CompilerParams(
            needs_layout_passes=False,
            use_tc_tiling_on_sc=False,
            disable_bounds_checks=True,
        ),
    )
    def sc_angles(
        packed_hbm, idx_hbm, out_hbm, idx_a, idx_b, rows_a, rows_b, out_v, sem_a, sem_b
    ):
        wid = lax.axis_index("s") * _NUM_CORES + lax.axis_index("c")
        iota3 = lax.iota(jnp.int32, _LANES) * 3
        col_vecs = [jnp.full((_LANES,), c, jnp.int32) for c in range(cols)]
        bvecs = [jnp.full((_LANES,), b, jnp.int32) for b in range(batch)]
        bufs = [(idx_a, rows_a, sem_a), (idx_b, rows_b, sem_b)]

        def stage_and_fire(s, idx_v, rows_v, sem):
            pltpu.sync_copy(
                idx_hbm.at[pl.ds(wid * tile_h + s * _ROWS_PER_SUB, _ROWS_PER_SUB)],
                idx_v,
            )
            return [
                pltpu.async_copy(
                    packed_hbm.at[idx_v.at[pl.ds(j * _IDX_BLK, _IDX_BLK)]],
                    rows_v.at[pl.ds(j * _IDX_BLK, _IDX_BLK)],
                    sem,
                )
                for j in range(_NBLK)
            ]

        def compute_sub(s, rows_v):
            def group_body(g, carry):
                qbase = g * (3 * _LANES) + iota3
                he0 = s * _ROWS_PER_SUB + g * (3 * _LANES) + iota3
                for b in range(batch):
                    P = [
                        [
                            plsc.load_gather(
                                rows_v, [qbase + vslot, col_vecs[b * 3 + c]]
                            )
                            for c in range(3)
                        ]
                        for vslot in range(3)
                    ]
                    e = [
                        [P[2][c] - P[0][c] for c in range(3)],
                        [P[0][c] - P[1][c] for c in range(3)],
                        [P[1][c] - P[2][c] for c in range(3)],
                    ]
                    n = [
                        e[i][0] * e[i][0] + e[i][1] * e[i][1] + e[i][2] * e[i][2]
                        for i in range(3)
                    ]
                    r = [_rsqrt(n[i], 2) for i in range(3)]
                    for j in range(3):
                        kj, ki = (j + 1) % 3, (j + 2) % 3
                        d = (
                            e[kj][0] * e[ki][0]
                            + e[kj][1] * e[ki][1]
                            + e[kj][2] * e[ki][2]
                        )
                        cos = -(d * r[kj]) * r[ki]
                        alpha = _acos(cos)
                        plsc.store_scatter(out_v, [bvecs[b], he0 + j], alpha)
                return carry

            lax.fori_loop(0, _GROUPS, group_body, 0)

        descs = stage_and_fire(0, *bufs[0])
        for s in range(nsub):
            nxt = stage_and_fire(s + 1, *bufs[(s + 1) % 2]) if s + 1 < nsub else None
            for d in descs:
                d.wait()
            compute_sub(s, bufs[s % 2][1])
            descs = nxt
        for b in range(batch):
            pltpu.sync_copy(
                out_v.at[b], out_hbm.at[pl.ds(b * h_pad + wid * tile_h, tile_h)]
            )

    out = sc_angles(packed, tails)
    return out.reshape(batch, h_pad)[:, : 3 * num_f]


# speedup vs baseline: 1.8471x; 1.8471x over previous
import functools
import math

import jax
import jax.numpy as jnp
from jax import lax
from jax.experimental import pallas as pl
from jax.experimental.pallas import tpu as pltpu
from jax.experimental.pallas import tpu_sc as plsc

_NUM_CORES = 2
_NUM_SUBCORES = 16
_NW = _NUM_CORES * _NUM_SUBCORES
_LANES = 16

_SUB_F = 640
_ROWS_PER_SUB = 3 * _SUB_F
_IDX_BLK = 128
_GROUPS = _SUB_F // _LANES

_SC_PARAMS = pltpu.CompilerParams(
    needs_layout_passes=False,
    use_tc_tiling_on_sc=False,
    disable_bounds_checks=True,
)
_MESH = dict(core_axis_name="c", subcore_axis_name="s")


def _rsqrt(x, iters):
    i = plsc.bitcast(x, jnp.int32)
    r = plsc.bitcast(jnp.int32(0x5F3759DF) - (i >> 1), jnp.float32)
    for _ in range(iters):
        r = r * (1.5 - 0.5 * x * r * r)
    return r


def _acos(x):
    ax = jnp.minimum(jnp.abs(x), 1.0)
    t = 1.0 - ax
    s = t * _rsqrt(t, 2)
    p = jnp.float32(-0.0187293)
    p = p * ax + 0.0742610
    p = p * ax - 0.2121144
    p = p * ax + 1.5707288
    a = s * p
    return jnp.where(x >= 0.0, a, jnp.float32(math.pi) - a)


def kernel(fs, faces):
    batch, num_v, _ = fs.shape
    num_f = faces.shape[0]
    cols = batch * 3
    assert cols <= _LANES and num_v % (8 * _NW) == 0

    per_round_f = _SUB_F * _NW
    nsub = -(-num_f // per_round_f)
    f_pad = nsub * per_round_f
    h_pad = 3 * f_pad
    tile_f = nsub * _SUB_F
    tile_h = 3 * tile_f
    vpt = num_v // _NW
    vgroups = vpt // _LANES

    planes = jnp.transpose(fs, (2, 0, 1)).reshape(cols, num_v)
    vcol = [
        jnp.pad(faces[:, k].astype(jnp.int32), (0, f_pad - num_f)) for k in range(3)
    ]

    @functools.partial(
        pl.kernel,
        out_type=jax.ShapeDtypeStruct((num_v * _LANES,), jnp.float32),
        mesh=plsc.VectorSubcoreMesh(**_MESH),
        scratch_types=[
            pltpu.VMEM((cols * vpt,), jnp.float32),
            pltpu.VMEM((vpt * _LANES,), jnp.float32),
        ],
        compiler_params=_SC_PARAMS,
    )
    def sc_pack(planes_hbm, packed_hbm, planes_v, rows_v):
        wid = lax.axis_index("s") * _NUM_CORES + lax.axis_index("c")
        v0 = wid * vpt
        iota = lax.iota(jnp.int32, _LANES)
        for p in range(cols):
            pltpu.sync_copy(
                planes_hbm.at[p, pl.ds(v0, vpt)], planes_v.at[pl.ds(p * vpt, vpt)]
            )

        def group_body(g, carry):
            ridx = (g * _LANES + iota) * _LANES
            for p in range(cols):
                x = planes_v[pl.ds(g * _LANES + p * vpt, _LANES)]
                plsc.store_scatter(rows_v, [ridx + p], x)
            return carry

        lax.fori_loop(0, vgroups, group_body, 0)
        pltpu.sync_copy(rows_v, packed_hbm.at[pl.ds(v0 * _LANES, vpt * _LANES)])

    packed = sc_pack(planes).reshape(num_v, _LANES)

    @functools.partial(
        pl.kernel,
        out_type=jax.ShapeDtypeStruct((batch * h_pad,), jnp.float32),
        mesh=plsc.VectorSubcoreMesh(**_MESH),
        scratch_types=[
            pltpu.VMEM((_ROWS_PER_SUB,), jnp.int32),
            pltpu.VMEM((_ROWS_PER_SUB,), jnp.int32),
            pltpu.VMEM((_ROWS_PER_SUB, _LANES), jnp.float32),
            pltpu.VMEM((_ROWS_PER_SUB, _LANES), jnp.float32),
            pltpu.VMEM((batch, tile_h), jnp.float32),
            pltpu.SemaphoreType.DMA,
            pltpu.SemaphoreType.DMA,
        ],
        compiler_params=_SC_PARAMS,
    )
    def sc_angles(
        packed_hbm, v0_hbm, v1_hbm, v2_hbm, out_hbm,
        idx_a, idx_b, rows_a, rows_b, out_v, sem_a, sem_b,
    ):
        wid = lax.axis_index("s") * _NUM_CORES + lax.axis_index("c")
        iota = lax.iota(jnp.int32, _LANES)
        iota3 = iota * 3
        col_vecs = [jnp.full((_LANES,), c, jnp.int32) for c in range(cols)]
        bvecs = [jnp.full((_LANES,), b, jnp.int32) for b in range(batch)]
        vk_hbm = [v0_hbm, v1_hbm, v2_hbm]
        bufs = [(idx_a, rows_a, sem_a), (idx_b, rows_b, sem_b)]

        def stage_and_fire(s, idx_v, rows_v, sem):
            f0 = wid * tile_f + s * _SUB_F
            for k in range(3):
                pltpu.sync_copy(
                    vk_hbm[k].at[pl.ds(f0, _SUB_F)],
                    idx_v.at[pl.ds(k * _SUB_F, _SUB_F)],
                )
            return [
                pltpu.async_copy(
                    packed_hbm.at[idx_v.at[pl.ds(j * _IDX_BLK, _IDX_BLK)]],
                    rows_v.at[pl.ds(j * _IDX_BLK, _IDX_BLK)],
                    sem,
                )
                for j in range(_ROWS_PER_SUB // _IDX_BLK)
            ]

        def compute_sub(s, rows_v):
            def group_body(g, carry):
                q0 = g * _LANES + iota
                he0 = s * _ROWS_PER_SUB + g * (3 * _LANES) + iota3
                for b in range(batch):
                    P = [
                        [
                            plsc.load_gather(
                                rows_v, [q0 + k * _SUB_F, col_vecs[c * batch + b]]
                            )
                            for c in range(3)
                        ]
                        for k in range(3)
                    ]
                    e = [
                        [P[2][c] - P[0][c] for c in range(3)],
                        [P[0][c] - P[1][c] for c in range(3)],
                        [P[1][c] - P[2][c] for c in range(3)],
                    ]
                    n = [
                        e[i][0] * e[i][0] + e[i][1] * e[i][1] + e[i][2] * e[i][2]
                        for i in range(3)
                    ]
                    r = [_rsqrt(n[i], 2) for i in range(3)]
                    for j in range(3):
                        kj, ki = (j + 1) % 3, (j + 2) % 3
                        d = (
                            e[kj][0] * e[ki][0]
                            + e[kj][1] * e[ki][1]
                            + e[kj][2] * e[ki][2]
                        )
                        cos = -(d * r[kj]) * r[ki]
                        alpha = _acos(cos)
                        plsc.store_scatter(out_v, [bvecs[b], he0 + j], alpha)
                return carry

            lax.fori_loop(0, _GROUPS, group_body, 0)

        descs = stage_and_fire(0, *bufs[0])
        for s in range(nsub):
            nxt = stage_and_fire(s + 1, *bufs[(s + 1) % 2]) if s + 1 < nsub else None
            for d in descs:
                d.wait()
            compute_sub(s, bufs[s % 2][1])
            descs = nxt
        for b in range(batch):
            pltpu.sync_copy(
                out_v.at[b], out_hbm.at[pl.ds(b * h_pad + wid * tile_h, tile_h)]
            )

    out = sc_angles(packed, *vcol)
    return out.reshape(batch, h_pad)[:, : 3 * num_f]


# speedup vs baseline: 1.9184x; 1.0386x over previous
import functools
import math

import jax
import jax.numpy as jnp
from jax import lax
from jax.experimental import pallas as pl
from jax.experimental.pallas import tpu as pltpu
from jax.experimental.pallas import tpu_sc as plsc

_NUM_CORES = 2
_NUM_SUBCORES = 16
_NW = _NUM_CORES * _NUM_SUBCORES
_LANES = 16

_SUB_F = 640
_ROWS_PER_SUB = 3 * _SUB_F
_IDX_BLK = 128
_GROUPS = _SUB_F // _LANES

_SC_PARAMS = pltpu.CompilerParams(
    needs_layout_passes=False,
    use_tc_tiling_on_sc=False,
    disable_bounds_checks=True,
)
_MESH = dict(core_axis_name="c", subcore_axis_name="s")


def _rsqrt(x, iters):
    i = plsc.bitcast(x, jnp.int32)
    r = plsc.bitcast(jnp.int32(0x5F3759DF) - (i >> 1), jnp.float32)
    for _ in range(iters):
        r = r * (1.5 - 0.5 * x * r * r)
    return r


def _acos(x):
    ax = jnp.minimum(jnp.abs(x), 1.0)
    t = 1.0 - ax
    s = t * _rsqrt(t, 1)
    p = jnp.float32(-0.0187293)
    p = p * ax + 0.0742610
    p = p * ax - 0.2121144
    p = p * ax + 1.5707288
    a = s * p
    return jnp.where(x >= 0.0, a, jnp.float32(math.pi) - a)


def kernel(fs, faces):
    batch, num_v, _ = fs.shape
    num_f = faces.shape[0]
    cols = batch * 3
    assert cols <= _LANES and num_v % (8 * _NW) == 0

    per_round_f = _SUB_F * _NW
    nsub = -(-num_f // per_round_f)
    f_pad = nsub * per_round_f
    h_pad = 3 * f_pad
    tile_f = nsub * _SUB_F
    tile_h = 3 * tile_f
    vpt = num_v // _NW
    vgroups = vpt // _LANES

    planes = jnp.transpose(fs, (2, 0, 1)).reshape(cols, num_v)
    vcol = [
        jnp.pad(faces[:, k].astype(jnp.int32), (0, f_pad - num_f)) for k in range(3)
    ]

    @functools.partial(
        pl.kernel,
        out_type=jax.ShapeDtypeStruct((num_v * _LANES,), jnp.float32),
        mesh=plsc.VectorSubcoreMesh(**_MESH),
        scratch_types=[
            pltpu.VMEM((cols * vpt,), jnp.float32),
            pltpu.VMEM((vpt * _LANES,), jnp.float32),
            pltpu.SemaphoreType.DMA,
        ],
        compiler_params=_SC_PARAMS,
    )
    def sc_pack(planes_hbm, packed_hbm, planes_v, rows_v, sem):
        wid = lax.axis_index("s") * _NUM_CORES + lax.axis_index("c")
        v0 = wid * vpt
        iota = lax.iota(jnp.int32, _LANES)
        descs = [
            pltpu.async_copy(
                planes_hbm.at[p, pl.ds(v0, vpt)],
                planes_v.at[pl.ds(p * vpt, vpt)],
                sem,
            )
            for p in range(cols)
        ]
        for d in descs:
            d.wait()

        @plsc.parallel_loop(0, vgroups)
        def group_body(g):
            ridx = (g * _LANES + iota) * _LANES
            for p in range(cols):
                x = planes_v[pl.ds(g * _LANES + p * vpt, _LANES)]
                plsc.store_scatter(rows_v, [ridx + p], x)
        pltpu.sync_copy(rows_v, packed_hbm.at[pl.ds(v0 * _LANES, vpt * _LANES)])

    packed = sc_pack(planes).reshape(num_v, _LANES)

    @functools.partial(
        pl.kernel,
        out_type=jax.ShapeDtypeStruct((batch * h_pad,), jnp.float32),
        mesh=plsc.VectorSubcoreMesh(**_MESH),
        scratch_types=[
            pltpu.VMEM((_ROWS_PER_SUB,), jnp.int32),
            pltpu.VMEM((_ROWS_PER_SUB,), jnp.int32),
            pltpu.VMEM((_ROWS_PER_SUB, _LANES), jnp.float32),
            pltpu.VMEM((_ROWS_PER_SUB, _LANES), jnp.float32),
            pltpu.VMEM((batch, tile_h), jnp.float32),
            pltpu.SemaphoreType.DMA,
            pltpu.SemaphoreType.DMA,
        ],
        compiler_params=_SC_PARAMS,
    )
    def sc_angles(
        packed_hbm, v0_hbm, v1_hbm, v2_hbm, out_hbm,
        idx_a, idx_b, rows_a, rows_b, out_v, sem_a, sem_b,
    ):
        wid = lax.axis_index("s") * _NUM_CORES + lax.axis_index("c")
        iota = lax.iota(jnp.int32, _LANES)
        iota3 = iota * 3
        col_vecs = [jnp.full((_LANES,), c, jnp.int32) for c in range(cols)]
        bvecs = [jnp.full((_LANES,), b, jnp.int32) for b in range(batch)]
        vk_hbm = [v0_hbm, v1_hbm, v2_hbm]
        bufs = [(idx_a, rows_a, sem_a), (idx_b, rows_b, sem_b)]

        def stage_and_fire(s, idx_v, rows_v, sem):
            f0 = wid * tile_f + s * _SUB_F
            for k in range(3):
                pltpu.sync_copy(
                    vk_hbm[k].at[pl.ds(f0, _SUB_F)],
                    idx_v.at[pl.ds(k * _SUB_F, _SUB_F)],
                )
            return [
                pltpu.async_copy(
                    packed_hbm.at[idx_v.at[pl.ds(j * _IDX_BLK, _IDX_BLK)]],
                    rows_v.at[pl.ds(j * _IDX_BLK, _IDX_BLK)],
                    sem,
                )
                for j in range(_ROWS_PER_SUB // _IDX_BLK)
            ]

        def compute_sub(s, rows_v):
            @plsc.parallel_loop(0, _GROUPS, unroll=2)
            def group_body(g):
                q0 = g * _LANES + iota
                he0 = s * _ROWS_PER_SUB + g * (3 * _LANES) + iota3
                for b in range(batch):
                    P = [
                        [
                            plsc.load_gather(
                                rows_v, [q0 + k * _SUB_F, col_vecs[c * batch + b]]
                            )
                            for c in range(3)
                        ]
                        for k in range(3)
                    ]
                    e = [
                        [P[2][c] - P[0][c] for c in range(3)],
                        [P[0][c] - P[1][c] for c in range(3)],
                        [P[1][c] - P[2][c] for c in range(3)],
                    ]
                    n = [
                        e[i][0] * e[i][0] + e[i][1] * e[i][1] + e[i][2] * e[i][2]
                        for i in range(3)
                    ]
                    r = [_rsqrt(n[i], 2) for i in range(3)]
                    for j in range(3):
                        kj, ki = (j + 1) % 3, (j + 2) % 3
                        d = (
                            e[kj][0] * e[ki][0]
                            + e[kj][1] * e[ki][1]
                            + e[kj][2] * e[ki][2]
                        )
                        cos = -(d * r[kj]) * r[ki]
                        alpha = _acos(cos)
                        plsc.store_scatter(out_v, [bvecs[b], he0 + j], alpha)

        descs = stage_and_fire(0, *bufs[0])
        for s in range(nsub):
            nxt = stage_and_fire(s + 1, *bufs[(s + 1) % 2]) if s + 1 < nsub else None
            for d in descs:
                d.wait()
            compute_sub(s, bufs[s % 2][1])
            descs = nxt
        for b in range(batch):
            pltpu.sync_copy(
                out_v.at[b], out_hbm.at[pl.ds(b * h_pad + wid * tile_h, tile_h)]
            )

    out = sc_angles(packed, *vcol)
    return out.reshape(batch, h_pad)[:, : 3 * num_f]
